# baseline (speedup 1.0000x reference)
import functools

import jax
import jax.numpy as jnp
from jax import lax
from jax.experimental import pallas as pl
from jax.experimental.pallas import tpu as pltpu
from jax.experimental.pallas import tpu_sc as plsc

N = 10000
D = 256
DP = D // 2
E = 160000
NC, NS = 2, 16
NW = NC * NS
EPW = E // NW
CB = 128
NR = (EPW + CB - 1) // CB
NG = CB // 16


def _normalize_body(z_ref, o_ref):
    x = z_ref[...]
    n2 = jnp.sum(x * x, axis=1, keepdims=True)
    inv = 1.0 / jnp.maximum(jnp.sqrt(n2), 1e-8)
    xn = (x * inv).astype(jnp.bfloat16)
    lo = lax.bitcast_convert_type(xn[:, :DP], jnp.uint16).astype(jnp.uint32)
    hi = lax.bitcast_convert_type(xn[:, DP:], jnp.uint16).astype(jnp.uint32)
    o_ref[...] = lax.bitcast_convert_type(lo | (hi << 16), jnp.int32)


def _normalize(z):
    BR = 1000
    return pl.pallas_call(
        _normalize_body,
        out_shape=jax.ShapeDtypeStruct((N, DP), jnp.int32),
        grid=(N // BR,),
        in_specs=[pl.BlockSpec((BR, D), lambda i: (i, 0))],
        out_specs=pl.BlockSpec((BR, DP), lambda i: (i, 0)),
    )(z)


_mesh = plsc.VectorSubcoreMesh(core_axis_name="c", subcore_axis_name="s")

_PERM_DN = lax.GatherDimensionNumbers(
    offset_dims=(), collapsed_slice_dims=(0,), start_index_map=(0,))


def _perm(v, idx):
    return lax.gather(v, idx[:, None], _PERM_DN, (1,),
                      mode=lax.GatherScatterMode.PROMISE_IN_BOUNDS)


@functools.partial(
    pl.kernel,
    out_type=jax.ShapeDtypeStruct((E,), jnp.float32),
    mesh=_mesh,
    scratch_types=[
        pltpu.VMEM((EPW,), jnp.int32),
        pltpu.VMEM((EPW,), jnp.int32),
        pltpu.VMEM((CB, DP), jnp.int32),
        pltpu.VMEM((CB, DP), jnp.int32),
        pltpu.VMEM((CB, DP), jnp.int32),
        pltpu.VMEM((CB, DP), jnp.int32),
        pltpu.VMEM((EPW,), jnp.float32),
        pltpu.VMEM((512,), jnp.float32),
        pltpu.SemaphoreType.DMA,
        pltpu.SemaphoreType.DMA,
    ],
    compiler_params=pltpu.CompilerParams(needs_layout_passes=False),
)
def _edge_sc(zn_hbm, ei_hbm, out_hbm,
             idx_a, idx_b, ra0, rb0, ra1, rb1, out_buf, accm, sem0, sem1):
    wid = lax.axis_index("s") * NC + lax.axis_index("c")
    wbase = wid * EPW
    lane = lax.iota(jnp.int32, 16)

    def off_of(r):
        return jnp.minimum(r * CB, EPW - CB)

    def issue(r, ra, rb, sem):
        off = off_of(r)
        pltpu.async_copy(zn_hbm.at[idx_a.at[pl.ds(off, CB)]], ra, sem)
        pltpu.async_copy(zn_hbm.at[idx_b.at[pl.ds(off, CB)]], rb, sem)

    def drain(ra, rb, sem):
        pltpu.make_async_copy(zn_hbm.at[pl.ds(0, CB)], ra, sem).wait()
        pltpu.make_async_copy(zn_hbm.at[pl.ds(0, CB)], rb, sem).wait()

    mask_lo = lane < 8

    def compute(r, ra, rb):
        off = off_of(r)

        def group_body(g, carry):
            abase = (g & 1) * 256
            for l in range(16):
                i = g * 16 + l
                acc = None
                for j in range(D // 32):
                    wa = plsc.bitcast(ra[i, pl.ds(j * 16, 16)], jnp.bfloat16)
                    wb = plsc.bitcast(rb[i, pl.ds(j * 16, 16)], jnp.bfloat16)
                    p = wa * wb
                    acc = p if acc is None else acc + p
                ae, ao = plsc.unpack(acc, format=plsc.PackFormat.INTERLEAVED)
                accm[pl.ds(abase + l * 16, 16)] = ae + ao
            rowbase = abase + lane * 16
            res = plsc.load_gather(accm, [rowbase])
            for j in range(1, 16):
                res = res + plsc.load_gather(accm, [rowbase + j])
            out_buf[pl.ds(off + g * 16, 16)] = 1.0 / (1.0 + jnp.exp(-res))
            return carry

        lax.fori_loop(0, NG, group_body, 0)

    pltpu.sync_copy(ei_hbm.at[pl.ds(wbase, EPW)], idx_a)
    pltpu.sync_copy(ei_hbm.at[pl.ds(E + wbase, EPW)], idx_b)
    issue(0, ra0, rb0, sem0)

    def pair_body(p, carry):
        r = p * 2
        drain(ra0, rb0, sem0)
        issue(r + 1, ra1, rb1, sem1)
        compute(r, ra0, rb0)
        drain(ra1, rb1, sem1)
        issue(r + 2, ra0, rb0, sem0)
        compute(r + 1, ra1, rb1)
        return carry

    lax.fori_loop(0, NR // 2, pair_body, 0)
    drain(ra0, rb0, sem0)
    pltpu.sync_copy(out_buf, out_hbm.at[pl.ds(wbase, EPW)])


def kernel(z, edge_index):
    ei = edge_index.astype(jnp.int32).reshape(-1)
    zp = _normalize(z)
    return _edge_sc(zp, ei)


# speedup vs baseline: 1.0015x; 1.0015x over previous
import functools

import jax
import jax.numpy as jnp
from jax import lax
from jax.experimental import pallas as pl
from jax.experimental.pallas import tpu as pltpu
from jax.experimental.pallas import tpu_sc as plsc

N = 10000
D = 256
DP = D // 2
E = 160000
NC, NS = 2, 16
NW = NC * NS
EPW = E // NW
CB = 128
NR = (EPW + CB - 1) // CB
NG = CB // 16


def _normalize_body(z_ref, o_ref):
    x = z_ref[...]
    n2 = jnp.sum(x * x, axis=1, keepdims=True)
    inv = 1.0 / jnp.maximum(jnp.sqrt(n2), 1e-8)
    xn = (x * inv).astype(jnp.bfloat16)
    lo = lax.bitcast_convert_type(xn[:, :DP], jnp.uint16).astype(jnp.uint32)
    hi = lax.bitcast_convert_type(xn[:, DP:], jnp.uint16).astype(jnp.uint32)
    o_ref[...] = lax.bitcast_convert_type(lo | (hi << 16), jnp.int32)


def _normalize(z):
    BR = 1000
    return pl.pallas_call(
        _normalize_body,
        out_shape=jax.ShapeDtypeStruct((N, DP), jnp.int32),
        grid=(N // BR,),
        in_specs=[pl.BlockSpec((BR, D), lambda i: (i, 0))],
        out_specs=pl.BlockSpec((BR, DP), lambda i: (i, 0)),
    )(z)


_mesh = plsc.VectorSubcoreMesh(core_axis_name="c", subcore_axis_name="s")

_PERM_DN = lax.GatherDimensionNumbers(
    offset_dims=(), collapsed_slice_dims=(0,), start_index_map=(0,))


def _perm(v, idx):
    return lax.gather(v, idx[:, None], _PERM_DN, (1,),
                      mode=lax.GatherScatterMode.PROMISE_IN_BOUNDS)


@functools.partial(
    pl.kernel,
    out_type=jax.ShapeDtypeStruct((E,), jnp.float32),
    mesh=_mesh,
    scratch_types=[
        pltpu.VMEM((EPW,), jnp.int32),
        pltpu.VMEM((EPW,), jnp.int32),
        pltpu.VMEM((CB, DP), jnp.int32),
        pltpu.VMEM((CB, DP), jnp.int32),
        pltpu.VMEM((CB, DP), jnp.int32),
        pltpu.VMEM((CB, DP), jnp.int32),
        pltpu.VMEM((EPW,), jnp.float32),
        pltpu.VMEM((256,), jnp.float32),
        pltpu.SemaphoreType.DMA,
        pltpu.SemaphoreType.DMA,
    ],
    compiler_params=pltpu.CompilerParams(needs_layout_passes=False),
)
def _edge_sc(zn_hbm, ei_hbm, out_hbm,
             idx_a, idx_b, ra0, rb0, ra1, rb1, out_buf, accm, sem0, sem1):
    wid = lax.axis_index("s") * NC + lax.axis_index("c")
    wbase = wid * EPW
    lane = lax.iota(jnp.int32, 16)

    def off_of(r):
        return jnp.minimum(r * CB, EPW - CB)

    def issue(r, ra, rb, sem):
        off = off_of(r)
        pltpu.async_copy(zn_hbm.at[idx_a.at[pl.ds(off, CB)]], ra, sem)
        pltpu.async_copy(zn_hbm.at[idx_b.at[pl.ds(off, CB)]], rb, sem)

    def drain(ra, rb, sem):
        pltpu.make_async_copy(zn_hbm.at[pl.ds(0, CB)], ra, sem).wait()
        pltpu.make_async_copy(zn_hbm.at[pl.ds(0, CB)], rb, sem).wait()

    mask_lo = lane < 8

    def compute(r, ra, rb):
        off = off_of(r)

        def group_body(g, carry):
            for l in range(16):
                i = g * 16 + l
                acc = None
                for j in range(D // 32):
                    wa = plsc.bitcast(ra[i, pl.ds(j * 16, 16)], jnp.bfloat16)
                    wb = plsc.bitcast(rb[i, pl.ds(j * 16, 16)], jnp.bfloat16)
                    p = wa * wb
                    acc = p if acc is None else acc + p
                ae, ao = plsc.unpack(acc, format=plsc.PackFormat.INTERLEAVED)
                accm[pl.ds(l * 16, 16)] = ae + ao
            rowbase = lane * 16
            res = plsc.load_gather(accm, [rowbase])
            for j in range(1, 16):
                res = res + plsc.load_gather(accm, [rowbase + j])
            out_buf[pl.ds(off + g * 16, 16)] = 1.0 / (1.0 + jnp.exp(-res))
            return carry

        lax.fori_loop(0, NG, group_body, 0)

    pltpu.sync_copy(ei_hbm.at[pl.ds(wbase, EPW)], idx_a)
    pltpu.sync_copy(ei_hbm.at[pl.ds(E + wbase, EPW)], idx_b)
    issue(0, ra0, rb0, sem0)

    def pair_body(p, carry):
        r = p * 2
        drain(ra0, rb0, sem0)
        issue(r + 1, ra1, rb1, sem1)
        compute(r, ra0, rb0)
        drain(ra1, rb1, sem1)
        issue(r + 2, ra0, rb0, sem0)
        compute(r + 1, ra1, rb1)
        return carry

    lax.fori_loop(0, NR // 2, pair_body, 0)
    drain(ra0, rb0, sem0)
    pltpu.sync_copy(out_buf, out_hbm.at[pl.ds(wbase, EPW)])


def kernel(z, edge_index):
    ei = edge_index.astype(jnp.int32).reshape(-1)
    zp = _normalize(z)
    return _edge_sc(zp, ei)
